# baseline (speedup 1.0000x reference)
import functools

import jax
import jax.numpy as jnp
from jax import lax
from jax.experimental import pallas as pl
from jax.experimental.pallas import tpu as pltpu
from jax.experimental.pallas import tpu_sc as plsc

E = 8
K = 2
D = 768
DFF = 768
T = 2048
P = K * T
TMG = 128
G = 39
GP = 40
RS = G * TMG

NC = 2
NS = 16
NW = NC * NS
PPW = P // NW
TPW = T // NW
WW = 128


def _router_kernel(x_ref, rw_ref, p_ref, w_ref, est_ref):
    x = x_ref[...]
    logits = lax.dot_general(x, rw_ref[...], (((1,), (1,)), ((), ())),
                             preferred_element_type=jnp.float32)
    lane = lax.broadcasted_iota(jnp.int32, (T, E), 1)
    l1 = jnp.max(logits, axis=-1, keepdims=True)
    e1 = jnp.min(jnp.where(logits == l1, lane, E), axis=-1, keepdims=True)
    masked = jnp.where(lane == e1, -jnp.inf, logits)
    l2 = jnp.max(masked, axis=-1, keepdims=True)
    e2 = jnp.min(jnp.where(masked == l2, lane, E), axis=-1, keepdims=True)
    z = jnp.exp(l2 - l1)
    w1 = 1.0 / (1.0 + z)
    w2 = 1.0 - w1

    e_all = jnp.concatenate([e1, e2], axis=0)
    w_all = jnp.concatenate([w1, w2], axis=0)
    onehot = (e_all == lax.broadcasted_iota(jnp.int32, (P, E), 1)
              ).astype(jnp.float32)

    C = 1024
    rr = lax.broadcasted_iota(jnp.int32, (C, C), 0)
    cc = lax.broadcasted_iota(jnp.int32, (C, C), 1)
    tri = (rr > cc).astype(jnp.float32)
    base = jnp.zeros((1, E), jnp.float32)
    rank_parts = []
    for c in range(P // C):
        oh = onehot[c * C:(c + 1) * C]
        r = lax.dot_general(tri, oh, (((1,), (0,)), ((), ())),
                            preferred_element_type=jnp.float32) + base
        rank_parts.append(jnp.sum(r * oh, axis=1, keepdims=True))
        base = base + jnp.sum(oh, axis=0, keepdims=True)
    rank_sel = jnp.concatenate(rank_parts, axis=0)

    counts = base
    tiles = jnp.floor((counts + (TMG - 1)) / TMG)
    up = (lax.broadcasted_iota(jnp.int32, (E, E), 0)
          < lax.broadcasted_iota(jnp.int32, (E, E), 1)).astype(jnp.float32)
    start = lax.dot_general(tiles, up, (((1,), (0,)), ((), ())),
                            preferred_element_type=jnp.float32)
    offp = start * TMG
    off_sel = jnp.sum(onehot * offp, axis=1, keepdims=True)
    p_ref[...] = (off_sel + rank_sel).astype(jnp.int32)
    w_ref[...] = w_all * jnp.ones((1, WW), jnp.float32)
    s_rows = lax.broadcasted_iota(jnp.int32, (GP, E), 0).astype(jnp.float32)
    est_ref[:, 0:1] = (jnp.sum((start <= s_rows).astype(jnp.int32), axis=1,
                               keepdims=True) - 1).astype(jnp.int32)
    total = jnp.sum(tiles, axis=1, keepdims=True)
    est_ref[:, 1:2] = (s_rows[:, 0:1] < total).astype(jnp.int32)


def _router(x, router_w):
    return pl.pallas_call(
        _router_kernel,
        out_shape=(
            jax.ShapeDtypeStruct((P, 1), jnp.int32),
            jax.ShapeDtypeStruct((P, WW), jnp.float32),
            jax.ShapeDtypeStruct((GP, 2), jnp.int32),
        ),
    )(x, router_w)


@functools.cache
def _sc_mesh():
    return plsc.VectorSubcoreMesh(core_axis_name="c", subcore_axis_name="s",
                                  num_cores=NC, num_subcores=NS)


def _sc_dispatch_body(x_hbm, p_hbm, w_hbm, xs_hbm, ws_hbm,
                      idx_v, rows_v, wrows_v, sem):
    wid = lax.axis_index("s") * NC + lax.axis_index("c")
    base = wid * PPW
    t0 = base % T
    pltpu.sync_copy(p_hbm.at[pl.ds(base, PPW)], idx_v)
    pltpu.sync_copy(x_hbm.at[pl.ds(t0, PPW)], rows_v)
    pltpu.sync_copy(w_hbm.at[pl.ds(base, PPW)], wrows_v)
    pltpu.async_copy(rows_v, xs_hbm.at[idx_v], sem).wait()
    pltpu.async_copy(wrows_v, ws_hbm.at[idx_v], sem).wait()


def _sc_dispatch(x, p_all, w_exp):
    return pl.kernel(
        _sc_dispatch_body,
        out_type=(
            jax.ShapeDtypeStruct((RS, D), jnp.float32),
            jax.ShapeDtypeStruct((RS, WW), jnp.float32),
        ),
        mesh=_sc_mesh(),
        scratch_types=[
            pltpu.VMEM((PPW,), jnp.int32),
            pltpu.VMEM((PPW, D), jnp.float32),
            pltpu.VMEM((PPW, WW), jnp.float32),
            pltpu.SemaphoreType.DMA,
        ],
    )(x, p_all, w_exp)


def _gemm_kernel(est_ref, xs_ref, ws_ref, wg_ref, wu_ref, wd_ref, o_ref):
    @pl.when(est_ref[pl.program_id(0), 1] == 1)
    def _active():
        xs = xs_ref[...]
        g = lax.dot_general(xs, wg_ref[0], (((1,), (1,)), ((), ())),
                            preferred_element_type=jnp.float32)
        u = lax.dot_general(xs, wu_ref[0], (((1,), (1,)), ((), ())),
                            preferred_element_type=jnp.float32)
        h = g * jax.nn.sigmoid(g) * u
        o = lax.dot_general(h, wd_ref[0], (((1,), (1,)), ((), ())),
                            preferred_element_type=jnp.float32)
        o_ref[...] = o * ws_ref[:, 0:1]


def _gemm(est, xs, ws, w_gate, w_up, w_down):
    grid_spec = pltpu.PrefetchScalarGridSpec(
        num_scalar_prefetch=1,
        grid=(G,),
        in_specs=[
            pl.BlockSpec((TMG, D), lambda s, est: (s, 0)),
            pl.BlockSpec((TMG, WW), lambda s, est: (s, 0)),
            pl.BlockSpec((1, DFF, D), lambda s, est: (est[s, 0], 0, 0)),
            pl.BlockSpec((1, DFF, D), lambda s, est: (est[s, 0], 0, 0)),
            pl.BlockSpec((1, D, DFF), lambda s, est: (est[s, 0], 0, 0)),
        ],
        out_specs=pl.BlockSpec((TMG, D), lambda s, est: (s, 0)),
    )
    return pl.pallas_call(
        _gemm_kernel,
        grid_spec=grid_spec,
        out_shape=jax.ShapeDtypeStruct((RS, D), jnp.float32),
        compiler_params=pltpu.CompilerParams(
            dimension_semantics=("arbitrary",),
        ),
    )(est, xs, ws, w_gate, w_up, w_down)


def _sc_combine_body(o_hbm, p_hbm, y_hbm, p0_v, p1_v, r0_v, r1_v, sem):
    wid = lax.axis_index("s") * NC + lax.axis_index("c")
    t0 = wid * TPW
    pltpu.sync_copy(p_hbm.at[pl.ds(t0, TPW)], p0_v)
    pltpu.sync_copy(p_hbm.at[pl.ds(T + t0, TPW)], p1_v)
    cp0 = pltpu.async_copy(o_hbm.at[p0_v], r0_v, sem)
    cp1 = pltpu.async_copy(o_hbm.at[p1_v], r1_v, sem)
    cp0.wait()
    cp1.wait()

    def row(i, _):
        for c in range(D // 16):
            sl = pl.ds(c * 16, 16)
            r0_v[i, sl] += r1_v[i, sl]
        return 0

    lax.fori_loop(0, TPW, row, 0)
    pltpu.sync_copy(r0_v, y_hbm.at[pl.ds(t0, TPW)])


def _sc_combine(o_sorted, p_all):
    return pl.kernel(
        _sc_combine_body,
        out_type=jax.ShapeDtypeStruct((T, D), jnp.float32),
        mesh=_sc_mesh(),
        scratch_types=[
            pltpu.VMEM((TPW,), jnp.int32),
            pltpu.VMEM((TPW,), jnp.int32),
            pltpu.VMEM((TPW, D), jnp.float32),
            pltpu.VMEM((TPW, D), jnp.float32),
            pltpu.SemaphoreType.DMA,
        ],
    )(o_sorted, p_all)


def kernel(hidden_states, router_w, w_gate, w_up, w_down):
    orig_shape = hidden_states.shape
    x = hidden_states.reshape(-1, orig_shape[-1])
    p2, w_exp, est = _router(x, router_w)
    p_all = p2.reshape(P)
    xs, ws = _sc_dispatch(x, p_all, w_exp)
    o_sorted = _gemm(est, xs, ws, w_gate, w_up, w_down)
    y = _sc_combine(o_sorted, p_all)
    return y.reshape(orig_shape)


# speedup vs baseline: 1.3133x; 1.3133x over previous
import functools

import jax
import jax.numpy as jnp
from jax import lax
from jax.experimental import pallas as pl
from jax.experimental.pallas import tpu as pltpu
from jax.experimental.pallas import tpu_sc as plsc

E = 8
K = 2
D = 768
DFF = 768
T = 2048
P = K * T
TMG = 256
G = 23
GP = 24
RS = G * TMG

NC = 2
NS = 16
NW = NC * NS
PPW = P // NW
TPW = T // NW
WW = 128


def _router_kernel(x_ref, rw_ref, p_ref, w_ref, est_ref):
    lg = lax.dot_general(rw_ref[...], x_ref[...], (((1,), (1,)), ((), ())),
                         preferred_element_type=jnp.float32)
    row = lax.broadcasted_iota(jnp.int32, (E, T), 0)
    l1 = jnp.max(lg, axis=0, keepdims=True)
    e1 = jnp.min(jnp.where(lg == l1, row, E), axis=0, keepdims=True)
    masked = jnp.where(row == e1, -jnp.inf, lg)
    l2 = jnp.max(masked, axis=0, keepdims=True)
    e2 = jnp.min(jnp.where(masked == l2, row, E), axis=0, keepdims=True)
    z = jnp.exp(l2 - l1)
    w1 = 1.0 / (1.0 + z)
    w2 = 1.0 - w1

    e_all = jnp.concatenate([e1, e2], axis=1)
    w_ref[...] = jnp.concatenate([w1, w2], axis=1)
    onehot = (e_all == lax.broadcasted_iota(jnp.int32, (E, P), 0)
              ).astype(jnp.float32)

    C = 1024
    rr = lax.broadcasted_iota(jnp.int32, (C, C), 0)
    cc = lax.broadcasted_iota(jnp.int32, (C, C), 1)
    triu = (rr < cc).astype(jnp.float32)
    base = jnp.zeros((E, 1), jnp.float32)
    rank_parts = []
    for c in range(P // C):
        oh = onehot[:, c * C:(c + 1) * C]
        r = lax.dot_general(oh, triu, (((1,), (0,)), ((), ())),
                            preferred_element_type=jnp.float32) + base
        rank_parts.append(jnp.sum(r * oh, axis=0, keepdims=True))
        base = base + jnp.sum(oh, axis=1, keepdims=True)
    rank_sel = jnp.concatenate(rank_parts, axis=1)

    counts = base
    tiles = jnp.floor((counts + (TMG - 1)) / TMG)
    tril = (lax.broadcasted_iota(jnp.int32, (E, E), 0)
            > lax.broadcasted_iota(jnp.int32, (E, E), 1)).astype(jnp.float32)
    start = lax.dot_general(tril, tiles, (((1,), (0,)), ((), ())),
                            preferred_element_type=jnp.float32)
    offp = start * TMG
    off_sel = jnp.sum(onehot * offp, axis=0, keepdims=True)
    p_ref[...] = (off_sel + rank_sel).astype(jnp.int32)
    s_cols = lax.broadcasted_iota(jnp.int32, (E, GP), 1).astype(jnp.float32)
    est_ref[0:1, :] = (jnp.sum((start <= s_cols).astype(jnp.int32), axis=0,
                               keepdims=True) - 1).astype(jnp.int32)
    total = jnp.sum(tiles, axis=0, keepdims=True)
    est_ref[1:2, :] = (s_cols[0:1, :] < total).astype(jnp.int32)


def _router(x, router_w):
    return pl.pallas_call(
        _router_kernel,
        out_shape=(
            jax.ShapeDtypeStruct((1, P), jnp.int32),
            jax.ShapeDtypeStruct((1, P), jnp.float32),
            jax.ShapeDtypeStruct((2, GP), jnp.int32),
        ),
    )(x, router_w)


@functools.cache
def _sc_mesh():
    return plsc.VectorSubcoreMesh(core_axis_name="c", subcore_axis_name="s",
                                  num_cores=NC, num_subcores=NS)


def _sc_dispatch_body(x_hbm, p_hbm, w_hbm, xs_hbm, ws_hbm,
                      idx_v, rows_v, wrows_v, sem):
    wid = lax.axis_index("s") * NC + lax.axis_index("c")
    base = wid * PPW
    t0 = base % T
    pltpu.sync_copy(p_hbm.at[pl.ds(base, PPW)], idx_v)
    pltpu.sync_copy(x_hbm.at[pl.ds(t0, PPW)], rows_v)
    pltpu.sync_copy(w_hbm.at[pl.ds(base, PPW)], wrows_v)
    pltpu.async_copy(rows_v, xs_hbm.at[idx_v], sem).wait()
    pltpu.async_copy(wrows_v, ws_hbm.at[idx_v], sem).wait()


def _sc_dispatch(x, p_all, w_exp):
    return pl.kernel(
        _sc_dispatch_body,
        out_type=(
            jax.ShapeDtypeStruct((RS, D), jnp.float32),
            jax.ShapeDtypeStruct((RS, WW), jnp.float32),
        ),
        mesh=_sc_mesh(),
        scratch_types=[
            pltpu.VMEM((PPW,), jnp.int32),
            pltpu.VMEM((PPW, D), jnp.float32),
            pltpu.VMEM((PPW, WW), jnp.float32),
            pltpu.SemaphoreType.DMA,
        ],
    )(x, p_all, w_exp)


def _gemm_kernel(est_ref, xs_ref, ws_ref, wg_ref, wu_ref, wd_ref, o_ref):
    @pl.when(est_ref[1, pl.program_id(0)] == 1)
    def _active():
        xs = xs_ref[...]
        g = lax.dot_general(xs, wg_ref[0], (((1,), (1,)), ((), ())),
                            preferred_element_type=jnp.float32)
        u = lax.dot_general(xs, wu_ref[0], (((1,), (1,)), ((), ())),
                            preferred_element_type=jnp.float32)
        h = g * jax.nn.sigmoid(g) * u
        o = lax.dot_general(h, wd_ref[0], (((1,), (1,)), ((), ())),
                            preferred_element_type=jnp.float32)
        o_ref[...] = o * ws_ref[:, 0:1]


def _gemm(est, xs, ws, w_gate, w_up, w_down):
    grid_spec = pltpu.PrefetchScalarGridSpec(
        num_scalar_prefetch=1,
        grid=(G,),
        in_specs=[
            pl.BlockSpec((TMG, D), lambda s, est: (s, 0)),
            pl.BlockSpec((TMG, WW), lambda s, est: (s, 0)),
            pl.BlockSpec((1, DFF, D), lambda s, est: (est[0, s], 0, 0)),
            pl.BlockSpec((1, DFF, D), lambda s, est: (est[0, s], 0, 0)),
            pl.BlockSpec((1, D, DFF), lambda s, est: (est[0, s], 0, 0)),
        ],
        out_specs=pl.BlockSpec((TMG, D), lambda s, est: (s, 0)),
    )
    return pl.pallas_call(
        _gemm_kernel,
        grid_spec=grid_spec,
        out_shape=jax.ShapeDtypeStruct((RS, D), jnp.float32),
        compiler_params=pltpu.CompilerParams(
            dimension_semantics=("arbitrary",),
        ),
    )(est, xs, ws, w_gate, w_up, w_down)


def _sc_combine_body(o_hbm, p_hbm, y_hbm, p0_v, p1_v, r0_v, r1_v, sem):
    wid = lax.axis_index("s") * NC + lax.axis_index("c")
    t0 = wid * TPW
    pltpu.sync_copy(p_hbm.at[pl.ds(t0, TPW)], p0_v)
    pltpu.sync_copy(p_hbm.at[pl.ds(T + t0, TPW)], p1_v)
    cp0 = pltpu.async_copy(o_hbm.at[p0_v], r0_v, sem)
    cp1 = pltpu.async_copy(o_hbm.at[p1_v], r1_v, sem)
    cp0.wait()
    cp1.wait()

    def row(i, _):
        for c in range(D // 16):
            sl = pl.ds(c * 16, 16)
            r0_v[i, sl] += r1_v[i, sl]
        return 0

    lax.fori_loop(0, TPW, row, 0)
    pltpu.sync_copy(r0_v, y_hbm.at[pl.ds(t0, TPW)])


def _sc_combine(o_sorted, p_all):
    return pl.kernel(
        _sc_combine_body,
        out_type=jax.ShapeDtypeStruct((T, D), jnp.float32),
        mesh=_sc_mesh(),
        scratch_types=[
            pltpu.VMEM((TPW,), jnp.int32),
            pltpu.VMEM((TPW,), jnp.int32),
            pltpu.VMEM((TPW, D), jnp.float32),
            pltpu.VMEM((TPW, D), jnp.float32),
            pltpu.SemaphoreType.DMA,
        ],
    )(o_sorted, p_all)


def kernel(hidden_states, router_w, w_gate, w_up, w_down):
    orig_shape = hidden_states.shape
    x = hidden_states.reshape(-1, orig_shape[-1])
    p2, w_row, est = _router(x, router_w)
    p_all = p2.reshape(P)
    w_exp = jnp.broadcast_to(w_row.reshape(P, 1), (P, WW))
    xs, ws = _sc_dispatch(x, p_all, w_exp)
    o_sorted = _gemm(est, xs, ws, w_gate, w_up, w_down)
    y = _sc_combine(o_sorted, p_all)
    return y.reshape(orig_shape)


# speedup vs baseline: 1.3155x; 1.0017x over previous
import functools

import jax
import jax.numpy as jnp
from jax import lax
from jax.experimental import pallas as pl
from jax.experimental.pallas import tpu as pltpu
from jax.experimental.pallas import tpu_sc as plsc

E = 8
K = 2
D = 768
DFF = 768
T = 2048
P = K * T
TMG = 256
G = 23
GP = 24
RS = G * TMG

NC = 2
NS = 16
NW = NC * NS
PPW = P // NW
TPW = T // NW
WW = 128


def _router_kernel(x_ref, rw_ref, p_ref, w_ref, est_ref):
    lg = lax.dot_general(rw_ref[...], x_ref[...], (((1,), (1,)), ((), ())),
                         preferred_element_type=jnp.float32)
    row = lax.broadcasted_iota(jnp.int32, (E, T), 0)
    l1 = jnp.max(lg, axis=0, keepdims=True)
    e1 = jnp.min(jnp.where(lg == l1, row, E), axis=0, keepdims=True)
    masked = jnp.where(row == e1, -jnp.inf, lg)
    l2 = jnp.max(masked, axis=0, keepdims=True)
    e2 = jnp.min(jnp.where(masked == l2, row, E), axis=0, keepdims=True)
    z = jnp.exp(l2 - l1)
    w1 = 1.0 / (1.0 + z)
    w2 = 1.0 - w1

    e_all = jnp.concatenate([e1, e2], axis=1)
    w_ref[...] = jnp.concatenate([w1, w2], axis=1)
    onehot = (e_all == lax.broadcasted_iota(jnp.int32, (E, P), 0)
              ).astype(jnp.float32)

    C = 1024
    rr = lax.broadcasted_iota(jnp.int32, (C, C), 0)
    cc = lax.broadcasted_iota(jnp.int32, (C, C), 1)
    triu = (rr < cc).astype(jnp.float32)
    base = jnp.zeros((E, 1), jnp.float32)
    rank_parts = []
    for c in range(P // C):
        oh = onehot[:, c * C:(c + 1) * C]
        r = lax.dot_general(oh, triu, (((1,), (0,)), ((), ())),
                            preferred_element_type=jnp.float32) + base
        rank_parts.append(jnp.sum(r * oh, axis=0, keepdims=True))
        base = base + jnp.sum(oh, axis=1, keepdims=True)
    rank_sel = jnp.concatenate(rank_parts, axis=1)

    counts = base
    tiles = jnp.floor((counts + (TMG - 1)) / TMG)
    tril = (lax.broadcasted_iota(jnp.int32, (E, E), 0)
            > lax.broadcasted_iota(jnp.int32, (E, E), 1)).astype(jnp.float32)
    start = lax.dot_general(tril, tiles, (((1,), (0,)), ((), ())),
                            preferred_element_type=jnp.float32)
    offp = start * TMG
    off_sel = jnp.sum(onehot * offp, axis=0, keepdims=True)
    p_ref[...] = (off_sel + rank_sel).astype(jnp.int32)
    s_cols = lax.broadcasted_iota(jnp.int32, (E, GP), 1).astype(jnp.float32)
    est_ref[0:1, :] = (jnp.sum((start <= s_cols).astype(jnp.int32), axis=0,
                               keepdims=True) - 1).astype(jnp.int32)
    total = jnp.sum(tiles, axis=0, keepdims=True)
    est_ref[1:2, :] = (s_cols[0:1, :] < total).astype(jnp.int32)


def _router(x, router_w):
    return pl.pallas_call(
        _router_kernel,
        out_shape=(
            jax.ShapeDtypeStruct((1, P), jnp.int32),
            jax.ShapeDtypeStruct((1, P), jnp.float32),
            jax.ShapeDtypeStruct((2, GP), jnp.int32),
        ),
    )(x, router_w)


@functools.cache
def _sc_mesh():
    return plsc.VectorSubcoreMesh(core_axis_name="c", subcore_axis_name="s",
                                  num_cores=NC, num_subcores=NS)


def _sc_dispatch_body(x_hbm, p_hbm, w_hbm, xs_hbm, ws_hbm,
                      idx_v, rows_v, wrows_v, sem):
    wid = lax.axis_index("s") * NC + lax.axis_index("c")
    base = wid * PPW
    t0 = base % T
    pltpu.sync_copy(p_hbm.at[pl.ds(base, PPW)], idx_v)
    pltpu.sync_copy(x_hbm.at[pl.ds(t0, PPW)], rows_v)
    pltpu.sync_copy(w_hbm.at[pl.ds(base, PPW)], wrows_v)
    cpx = pltpu.async_copy(rows_v, xs_hbm.at[idx_v], sem)
    cpw = pltpu.async_copy(wrows_v, ws_hbm.at[idx_v], sem)
    cpx.wait()
    cpw.wait()


def _sc_dispatch(x, p_all, w_exp):
    return pl.kernel(
        _sc_dispatch_body,
        out_type=(
            jax.ShapeDtypeStruct((RS, D), jnp.float32),
            jax.ShapeDtypeStruct((RS, WW), jnp.float32),
        ),
        mesh=_sc_mesh(),
        scratch_types=[
            pltpu.VMEM((PPW,), jnp.int32),
            pltpu.VMEM((PPW, D), jnp.float32),
            pltpu.VMEM((PPW, WW), jnp.float32),
            pltpu.SemaphoreType.DMA,
        ],
    )(x, p_all, w_exp)


def _gemm_kernel(est_ref, xs_ref, ws_ref, wg_ref, wu_ref, wd_ref, o_ref):
    @pl.when(est_ref[1, pl.program_id(0)] == 1)
    def _active():
        xs = xs_ref[...]
        g = lax.dot_general(xs, wg_ref[0], (((1,), (1,)), ((), ())),
                            preferred_element_type=jnp.float32)
        u = lax.dot_general(xs, wu_ref[0], (((1,), (1,)), ((), ())),
                            preferred_element_type=jnp.float32)
        h = g * jax.nn.sigmoid(g) * u
        o = lax.dot_general(h, wd_ref[0], (((1,), (1,)), ((), ())),
                            preferred_element_type=jnp.float32)
        o_ref[...] = o * ws_ref[:, 0:1]


def _gemm(est, xs, ws, w_gate, w_up, w_down):
    grid_spec = pltpu.PrefetchScalarGridSpec(
        num_scalar_prefetch=1,
        grid=(G,),
        in_specs=[
            pl.BlockSpec((TMG, D), lambda s, est: (s, 0)),
            pl.BlockSpec((TMG, WW), lambda s, est: (s, 0)),
            pl.BlockSpec((1, DFF, D), lambda s, est: (est[0, s], 0, 0)),
            pl.BlockSpec((1, DFF, D), lambda s, est: (est[0, s], 0, 0)),
            pl.BlockSpec((1, D, DFF), lambda s, est: (est[0, s], 0, 0)),
        ],
        out_specs=pl.BlockSpec((TMG, D), lambda s, est: (s, 0)),
    )
    return pl.pallas_call(
        _gemm_kernel,
        grid_spec=grid_spec,
        out_shape=jax.ShapeDtypeStruct((RS, D), jnp.float32),
        compiler_params=pltpu.CompilerParams(
            dimension_semantics=("arbitrary",),
        ),
    )(est, xs, ws, w_gate, w_up, w_down)


def _sc_combine_body(o_hbm, p_hbm, y_hbm, p0_v, p1_v, r0_v, r1_v, sem):
    wid = lax.axis_index("s") * NC + lax.axis_index("c")
    t0 = wid * TPW
    pltpu.sync_copy(p_hbm.at[pl.ds(t0, TPW)], p0_v)
    pltpu.sync_copy(p_hbm.at[pl.ds(T + t0, TPW)], p1_v)
    cp0 = pltpu.async_copy(o_hbm.at[p0_v], r0_v, sem)
    cp1 = pltpu.async_copy(o_hbm.at[p1_v], r1_v, sem)
    cp0.wait()
    cp1.wait()

    def row(i, _):
        for c in range(D // 16):
            sl = pl.ds(c * 16, 16)
            r0_v[i, sl] += r1_v[i, sl]
        return 0

    lax.fori_loop(0, TPW, row, 0)
    pltpu.sync_copy(r0_v, y_hbm.at[pl.ds(t0, TPW)])


def _sc_combine(o_sorted, p_all):
    return pl.kernel(
        _sc_combine_body,
        out_type=jax.ShapeDtypeStruct((T, D), jnp.float32),
        mesh=_sc_mesh(),
        scratch_types=[
            pltpu.VMEM((TPW,), jnp.int32),
            pltpu.VMEM((TPW,), jnp.int32),
            pltpu.VMEM((TPW, D), jnp.float32),
            pltpu.VMEM((TPW, D), jnp.float32),
            pltpu.SemaphoreType.DMA,
        ],
    )(o_sorted, p_all)


def kernel(hidden_states, router_w, w_gate, w_up, w_down):
    orig_shape = hidden_states.shape
    x = hidden_states.reshape(-1, orig_shape[-1])
    p2, w_row, est = _router(x, router_w)
    p_all = p2.reshape(P)
    w_exp = jnp.broadcast_to(w_row.reshape(P, 1), (P, WW))
    xs, ws = _sc_dispatch(x, p_all, w_exp)
    o_sorted = _gemm(est, xs, ws, w_gate, w_up, w_down)
    y = _sc_combine(o_sorted, p_all)
    return y.reshape(orig_shape)
